# baseline (speedup 1.0000x reference)
import functools

import jax
import jax.numpy as jnp
from jax import lax
from jax.experimental import pallas as pl
from jax.experimental.pallas import tpu as pltpu
from jax.experimental.pallas import tpu_sc as plsc

V = 100000
D = 64
B = 1024
L = 20

NUM_CORES = 2
NUM_SUBCORES = 16
NW = NUM_CORES * NUM_SUBCORES
DPW = D // NW
LANES = 16
_BGROUPS = B // LANES

_sc_mesh = plsc.VectorSubcoreMesh(core_axis_name="c", subcore_axis_name="s")


_SLAB_CHUNKS = 4
_SLAB_CHUNK = V // _SLAB_CHUNKS


@functools.partial(
    pl.kernel,
    mesh=_sc_mesh,
    out_type=jax.ShapeDtypeStruct((D, B), jnp.float32),
    scratch_types=[
        pltpu.VMEM((L, B), jnp.int32),
        pltpu.VMEM((V,), jnp.float32),
        pltpu.VMEM((B,), jnp.float32),
        pltpu.SemaphoreType.DMA,
    ],
    compiler_params=pltpu.CompilerParams(needs_layout_passes=False),
)
def _pool_sc(idx_hbm, table_hbm, out_hbm, idx_v, slab_v, pool_v, sem):
    wid = lax.axis_index("s") * NUM_CORES + lax.axis_index("c")
    idx_cp = pltpu.async_copy(idx_hbm, idx_v, sem)
    scale = jnp.float32(1.0 / L)
    out_cp = None
    for d_off in range(DPW):
        d_row = wid * DPW + d_off
        slab_cp = pltpu.async_copy(table_hbm.at[d_row], slab_v, sem)
        if d_off == 0:
            idx_cp.wait()
        else:
            out_cp.wait()
        slab_cp.wait()

        def body(g, carry):
            for u in range(2):
                sl = pl.ds((g * 2 + u) * LANES, LANES)
                acc = jnp.zeros((LANES,), jnp.float32)
                for l_i in range(L):
                    acc = acc + plsc.load_gather(slab_v, [idx_v[l_i, sl]])
                pool_v[sl] = acc * scale
            return carry

        lax.fori_loop(0, _BGROUPS // 2, body, 0)
        out_cp = pltpu.async_copy(pool_v, out_hbm.at[d_row], sem)
    out_cp.wait()


VBLK = 4096
_VGRID = pl.cdiv(V, VBLK)


def _decode_body(p_ref, wt_ref, bias_ref, o_ref):
    acc = lax.dot_general(
        wt_ref[...],
        p_ref[...],
        (((0,), (0,)), ((), ())),
        preferred_element_type=jnp.float32,
    )
    ones = jnp.ones((1, B), jnp.float32)
    bias2d = lax.dot_general(
        bias_ref[...],
        ones,
        (((0,), (0,)), ((), ())),
        preferred_element_type=jnp.float32,
    )
    o_ref[...] = acc + bias2d


_decode = pl.pallas_call(
    _decode_body,
    grid=(_VGRID,),
    in_specs=[
        pl.BlockSpec((D, B), lambda i: (0, 0)),
        pl.BlockSpec((D, VBLK), lambda i: (0, i)),
        pl.BlockSpec((1, VBLK), lambda i: (0, i)),
    ],
    out_specs=pl.BlockSpec((VBLK, B), lambda i: (i, 0)),
    out_shape=jax.ShapeDtypeStruct((V, B), jnp.float32),
    compiler_params=pltpu.CompilerParams(vmem_limit_bytes=100 * 1024 * 1024),
)


def kernel(contextTsr, emb_table, W, b):
    pooled_t = _pool_sc(contextTsr.T.astype(jnp.int32), emb_table.T)
    out_t = _decode(pooled_t, W.T, b.reshape(1, V))
    return out_t.T


# speedup vs baseline: 1.0020x; 1.0020x over previous
import functools

import jax
import jax.numpy as jnp
from jax import lax
from jax.experimental import pallas as pl
from jax.experimental.pallas import tpu as pltpu
from jax.experimental.pallas import tpu_sc as plsc

V = 100000
D = 64
B = 1024
L = 20

NUM_CORES = 2
NUM_SUBCORES = 16
NW = NUM_CORES * NUM_SUBCORES
DPW = D // NW
LANES = 16
_BGROUPS = B // LANES

_sc_mesh = plsc.VectorSubcoreMesh(core_axis_name="c", subcore_axis_name="s")


_SLAB_CHUNKS = 4
_SLAB_CHUNK = V // _SLAB_CHUNKS


@functools.partial(
    pl.kernel,
    mesh=_sc_mesh,
    out_type=jax.ShapeDtypeStruct((D, B), jnp.float32),
    scratch_types=[
        pltpu.VMEM((L, B), jnp.int32),
        pltpu.VMEM((V,), jnp.float32),
        pltpu.VMEM((B,), jnp.float32),
        pltpu.SemaphoreType.DMA,
    ],
    compiler_params=pltpu.CompilerParams(needs_layout_passes=False),
)
def _pool_sc(idx_hbm, table_hbm, out_hbm, idx_v, slab_v, pool_v, sem):
    wid = lax.axis_index("s") * NUM_CORES + lax.axis_index("c")
    idx_cp = pltpu.async_copy(idx_hbm, idx_v, sem)
    scale = jnp.float32(1.0 / L)
    out_cp = None
    for d_off in range(DPW):
        d_row = wid * DPW + d_off
        slab_cp = pltpu.async_copy(table_hbm.at[d_row], slab_v, sem)
        if d_off == 0:
            idx_cp.wait()
        else:
            out_cp.wait()
        slab_cp.wait()

        @plsc.parallel_loop(0, _BGROUPS, unroll=2)
        def _(g):
            sl = pl.ds(g * LANES, LANES)
            acc = jnp.zeros((LANES,), jnp.float32)
            for l_i in range(L):
                acc = acc + plsc.load_gather(slab_v, [idx_v[l_i, sl]])
            pool_v[sl] = acc * scale
        out_cp = pltpu.async_copy(pool_v, out_hbm.at[d_row], sem)
    out_cp.wait()


VBLK = 4096
_VGRID = pl.cdiv(V, VBLK)


def _decode_body(p_ref, wt_ref, bias_ref, o_ref):
    acc = lax.dot_general(
        wt_ref[...],
        p_ref[...],
        (((0,), (0,)), ((), ())),
        preferred_element_type=jnp.float32,
    )
    ones = jnp.ones((1, B), jnp.float32)
    bias2d = lax.dot_general(
        bias_ref[...],
        ones,
        (((0,), (0,)), ((), ())),
        preferred_element_type=jnp.float32,
    )
    o_ref[...] = acc + bias2d


_decode = pl.pallas_call(
    _decode_body,
    grid=(_VGRID,),
    in_specs=[
        pl.BlockSpec((D, B), lambda i: (0, 0)),
        pl.BlockSpec((D, VBLK), lambda i: (0, i)),
        pl.BlockSpec((1, VBLK), lambda i: (0, i)),
    ],
    out_specs=pl.BlockSpec((VBLK, B), lambda i: (i, 0)),
    out_shape=jax.ShapeDtypeStruct((V, B), jnp.float32),
    compiler_params=pltpu.CompilerParams(vmem_limit_bytes=100 * 1024 * 1024),
)


def kernel(contextTsr, emb_table, W, b):
    pooled_t = _pool_sc(contextTsr.T.astype(jnp.int32), emb_table.T)
    out_t = _decode(pooled_t, W.T, b.reshape(1, V))
    return out_t.T
